# Initial kernel scaffold; baseline (speedup 1.0000x reference)
import jax
import jax.numpy as jnp
from jax.experimental import pallas as pl


def kernel(x_user, x_item, edge_index_u2i, edge_index_i2u, W_l_u2i, b_l_u2i, W_r_u2i, W_l_i2u, b_l_i2u, W_r_i2u, W_user, b_user, W_item, b_item):
    raise NotImplementedError("write your pallas kernel here")


# speedup vs baseline: 5.0999x; 5.0999x over previous
import functools

import jax
import jax.numpy as jnp
from jax import lax
from jax.experimental import pallas as pl
from jax.experimental.pallas import tpu as pltpu
from jax.experimental.pallas import tpu_sc as plsc

_N = 10000
_D = 128
_E = 320000
_NC, _NS = 2, 16
_CW = 128
_EPS = 20480
_CH = _EPS // _CW
_STG = 16
_NPAD = 10240
_RPS = _NPAD // _NS
_DUMMY = _N


def _accumulate(x_src, src_hbm, dst_hbm, acc_hbm, cnt_hbm, zeros_hbm,
                zeros1d_hbm, acc_sh, src_stg, dst_stg, rows_v, cnt_v, sem, s):
    r0 = s * _RPS
    pltpu.sync_copy(zeros_hbm, acc_sh.at[pl.ds(r0, _RPS), :])
    pltpu.sync_copy(zeros1d_hbm, cnt_v)
    plsc.subcore_barrier()

    ones16 = jnp.ones((16,), jnp.float32)

    def group(g, carry):
        pltpu.sync_copy(src_hbm.at[s, pl.ds(g * _STG, _STG)], src_stg)
        pltpu.sync_copy(dst_hbm.at[s, pl.ds(g * _STG, _STG)], dst_stg)

        def chunk(j, carry2):
            pltpu.async_copy(x_src.at[src_stg.at[j]], rows_v, sem).wait()
            pltpu.sync_copy(rows_v, acc_sh.at[dst_stg.at[j]], add=True)
            for l in range(_CW // 16):
                d16 = dst_stg[j, pl.ds(l * 16, 16)]
                plsc.addupdate_scatter(cnt_v, (d16,), ones16)
            return carry2

        return lax.fori_loop(0, _STG, chunk, carry)

    lax.fori_loop(0, _CH // _STG, group, 0)
    plsc.subcore_barrier()
    pltpu.sync_copy(acc_sh.at[pl.ds(r0, _RPS), :], acc_hbm.at[pl.ds(r0, _RPS), :])
    pltpu.sync_copy(cnt_v, cnt_hbm.at[s])


def _sc_body(x_user, x_item, src_u2i, dst_u2i, src_i2u, dst_i2u,
             zeros_hbm, zeros1d_hbm, acc_out, cnt_out,
             acc_sh, src_stg, dst_stg, rows_v, cnt_v, sem):
    c = lax.axis_index("c")
    s = lax.axis_index("s")

    @pl.when(c == 0)
    def _():
        _accumulate(x_user, src_u2i, dst_u2i, acc_out.at[0], cnt_out.at[0],
                    zeros_hbm, zeros1d_hbm, acc_sh, src_stg, dst_stg,
                    rows_v, cnt_v, sem, s)

    @pl.when(c == 1)
    def _():
        _accumulate(x_item, src_i2u, dst_i2u, acc_out.at[1], cnt_out.at[1],
                    zeros_hbm, zeros1d_hbm, acc_sh, src_stg, dst_stg,
                    rows_v, cnt_v, sem, s)


@functools.cache
def _sc_segment_sum():
    return pl.kernel(
        _sc_body,
        out_type=(
            jax.ShapeDtypeStruct((_NC, _NPAD, _D), jnp.float32),
            jax.ShapeDtypeStruct((_NC, _NS, _NPAD), jnp.float32),
        ),
        mesh=plsc.VectorSubcoreMesh(core_axis_name="c", subcore_axis_name="s",
                                    num_cores=_NC, num_subcores=_NS),
        scratch_types=[
            pltpu.VMEM_SHARED((_NPAD, _D), jnp.float32),
            pltpu.VMEM((_STG, _CW), jnp.int32),
            pltpu.VMEM((_STG, _CW), jnp.int32),
            pltpu.VMEM((_CW, _D), jnp.float32),
            pltpu.VMEM((_NPAD,), jnp.float32),
            pltpu.SemaphoreType.DMA,
        ],
        compiler_params=pltpu.CompilerParams(needs_layout_passes=False),
    )


_BLK = 2560


def _dense_body(acc_ref, cnt_ref, x_ref, wl_ref, bl_ref, wr_ref, wo_ref, bo_ref,
                out_ref):
    cnt = jnp.maximum(jnp.sum(cnt_ref[0], axis=1, keepdims=True), 1.0)
    mean = acc_ref[0] / cnt
    dims = (((1,), (1,)), ((), ()))
    h = (lax.dot_general(mean, wl_ref[0], dims, preferred_element_type=jnp.float32)
         + bl_ref[0]
         + lax.dot_general(x_ref[0], wr_ref[0], dims, preferred_element_type=jnp.float32))
    out = lax.dot_general(h, wo_ref[0], dims, preferred_element_type=jnp.float32) + bo_ref[0]
    out_ref[...] = jnp.maximum(out, 0.0)[None]


def _dense_stage(acc, cnt_t, x, wl, bl, wr, wo, bo):
    spec_rows = pl.BlockSpec((1, _BLK, _D), lambda t, i: (t, i, 0))
    spec_cnt = pl.BlockSpec((1, _BLK, _NS), lambda t, i: (t, i, 0))
    spec_w = pl.BlockSpec((1, _D, _D), lambda t, i: (t, 0, 0))
    spec_b = pl.BlockSpec((1, 1, _D), lambda t, i: (t, 0, 0))
    return pl.pallas_call(
        _dense_body,
        grid=(2, _NPAD // _BLK),
        in_specs=[spec_rows, spec_cnt, spec_rows, spec_w, spec_b, spec_w,
                  spec_w, spec_b],
        out_specs=spec_rows,
        out_shape=jax.ShapeDtypeStruct((2, _NPAD, _D), jnp.float32),
    )(acc, cnt_t, x, wl, bl, wr, wo, bo)


def _pad_edges(edge_index):
    src = edge_index[0].reshape(_NS, _E // _NS)
    dst = edge_index[1].reshape(_NS, _E // _NS)
    pad = _EPS - _E // _NS
    src = jnp.concatenate(
        [src, jnp.zeros((_NS, pad), jnp.int32)], axis=1).reshape(_NS, _CH, _CW)
    dst = jnp.concatenate(
        [dst, jnp.full((_NS, pad), _DUMMY, jnp.int32)], axis=1).reshape(_NS, _CH, _CW)
    return src, dst


def kernel(x_user, x_item, edge_index_u2i, edge_index_i2u,
           W_l_u2i, b_l_u2i, W_r_u2i, W_l_i2u, b_l_i2u, W_r_i2u,
           W_user, b_user, W_item, b_item):
    src_u2i, dst_u2i = _pad_edges(edge_index_u2i)
    src_i2u, dst_i2u = _pad_edges(edge_index_i2u)
    zeros = jnp.zeros((_RPS, _D), jnp.float32)
    zeros1d = jnp.zeros((_NPAD,), jnp.float32)

    acc, cnt = _sc_segment_sum()(x_user, x_item, src_u2i, dst_u2i,
                                 src_i2u, dst_i2u, zeros, zeros1d)

    cnt_t = jnp.transpose(cnt, (0, 2, 1))
    x = jnp.zeros((2, _NPAD, _D), jnp.float32).at[:, :_N, :].set(
        jnp.stack([x_item, x_user]))
    wl = jnp.stack([W_l_u2i, W_l_i2u])
    bl = jnp.stack([b_l_u2i, b_l_i2u])[:, None, :]
    wr = jnp.stack([W_r_u2i, W_r_i2u])
    wo = jnp.stack([W_item, W_user])
    bo = jnp.stack([b_item, b_user])[:, None, :]
    out = _dense_stage(acc, cnt_t, x, wl, bl, wr, wo, bo)
    return (out[1, :_N], out[0, :_N])


# speedup vs baseline: 6.1089x; 1.1978x over previous
import functools

import jax
import jax.numpy as jnp
from jax import lax
from jax.experimental import pallas as pl
from jax.experimental.pallas import tpu as pltpu
from jax.experimental.pallas import tpu_sc as plsc

_N = 10000
_D = 128
_E = 320000
_NC, _NS = 2, 16
_CW = 128
_EPS = 20480
_CH = _EPS // _CW
_STG = 16
_NPAD = 10240
_RPS = _NPAD // _NS
_DUMMY = _N


def _accumulate(x_src, src_hbm, dst_hbm, acc_hbm, cnt_hbm, zeros_hbm,
                zeros1d_hbm, acc_sh, src_stg, dst_stg, rows_a, rows_b, cnt_v,
                sem_ga, sem_gb, sem_sa, sem_sb, s):
    r0 = s * _RPS
    pltpu.sync_copy(zeros_hbm, acc_sh.at[pl.ds(r0, _RPS), :])
    pltpu.sync_copy(zeros1d_hbm, cnt_v)
    plsc.subcore_barrier()

    ones16 = jnp.ones((16,), jnp.float32)
    dummy_rows = x_src.at[pl.ds(0, _CW), :]

    def counts(j):
        for l in range(_CW // 16):
            d16 = dst_stg[j, pl.ds(l * 16, 16)]
            plsc.addupdate_scatter(cnt_v, (d16,), ones16)

    def wait_gather(buf, sem):
        pltpu.make_async_copy(dummy_rows, buf, sem).wait()

    def wait_scatter(buf, sem):
        pltpu.make_async_copy(buf, acc_sh.at[pl.ds(0, _CW), :], sem).wait()

    npairs = _STG // 2

    def group(g, carry):
        pltpu.sync_copy(src_hbm.at[s, pl.ds(g * _STG, _STG)], src_stg)
        pltpu.sync_copy(dst_hbm.at[s, pl.ds(g * _STG, _STG)], dst_stg)
        pltpu.async_copy(x_src.at[src_stg.at[0]], rows_a, sem_ga)

        def pair(p, carry2):
            j0 = 2 * p
            @pl.when(p > 0)
            def _():
                wait_scatter(rows_b, sem_sb)

            pltpu.async_copy(x_src.at[src_stg.at[j0 + 1]], rows_b, sem_gb)
            wait_gather(rows_a, sem_ga)
            pltpu.async_copy(rows_a, acc_sh.at[dst_stg.at[j0]], sem_sa, add=True)
            counts(j0)
            wait_gather(rows_b, sem_gb)
            wait_scatter(rows_a, sem_sa)

            @pl.when(p < npairs - 1)
            def _():
                pltpu.async_copy(x_src.at[src_stg.at[j0 + 2]], rows_a, sem_ga)

            pltpu.async_copy(rows_b, acc_sh.at[dst_stg.at[j0 + 1]], sem_sb, add=True)
            counts(j0 + 1)
            return carry2

        carry = lax.fori_loop(0, npairs, pair, carry)
        wait_scatter(rows_b, sem_sb)
        return carry

    lax.fori_loop(0, _CH // _STG, group, 0)
    plsc.subcore_barrier()
    pltpu.sync_copy(acc_sh.at[pl.ds(r0, _RPS), :], acc_hbm.at[pl.ds(r0, _RPS), :])
    pltpu.sync_copy(cnt_v, cnt_hbm.at[s])


def _sc_body(x_user, x_item, src_u2i, dst_u2i, src_i2u, dst_i2u,
             zeros_hbm, zeros1d_hbm, acc_out, cnt_out,
             acc_sh, src_stg, dst_stg, rows_a, rows_b, cnt_v,
             sem_ga, sem_gb, sem_sa, sem_sb):
    c = lax.axis_index("c")
    s = lax.axis_index("s")

    @pl.when(c == 0)
    def _():
        _accumulate(x_user, src_u2i, dst_u2i, acc_out.at[0], cnt_out.at[0],
                    zeros_hbm, zeros1d_hbm, acc_sh, src_stg, dst_stg,
                    rows_a, rows_b, cnt_v, sem_ga, sem_gb, sem_sa, sem_sb, s)

    @pl.when(c == 1)
    def _():
        _accumulate(x_item, src_i2u, dst_i2u, acc_out.at[1], cnt_out.at[1],
                    zeros_hbm, zeros1d_hbm, acc_sh, src_stg, dst_stg,
                    rows_a, rows_b, cnt_v, sem_ga, sem_gb, sem_sa, sem_sb, s)


@functools.cache
def _sc_segment_sum():
    return pl.kernel(
        _sc_body,
        out_type=(
            jax.ShapeDtypeStruct((_NC, _NPAD, _D), jnp.float32),
            jax.ShapeDtypeStruct((_NC, _NS, _NPAD), jnp.float32),
        ),
        mesh=plsc.VectorSubcoreMesh(core_axis_name="c", subcore_axis_name="s",
                                    num_cores=_NC, num_subcores=_NS),
        scratch_types=[
            pltpu.VMEM_SHARED((_NPAD, _D), jnp.float32),
            pltpu.VMEM((_STG, _CW), jnp.int32),
            pltpu.VMEM((_STG, _CW), jnp.int32),
            pltpu.VMEM((_CW, _D), jnp.float32),
            pltpu.VMEM((_CW, _D), jnp.float32),
            pltpu.VMEM((_NPAD,), jnp.float32),
            pltpu.SemaphoreType.DMA,
            pltpu.SemaphoreType.DMA,
            pltpu.SemaphoreType.DMA,
            pltpu.SemaphoreType.DMA,
        ],
        compiler_params=pltpu.CompilerParams(needs_layout_passes=False),
    )


_BLK = 2560


def _dense_body(acc_ref, cnt_ref, x_ref, wl_ref, bl_ref, wr_ref, wo_ref, bo_ref,
                out_ref):
    cnt = jnp.maximum(jnp.sum(cnt_ref[0], axis=1, keepdims=True), 1.0)
    mean = acc_ref[0] / cnt
    dims = (((1,), (1,)), ((), ()))
    h = (lax.dot_general(mean, wl_ref[0], dims, preferred_element_type=jnp.float32)
         + bl_ref[0]
         + lax.dot_general(x_ref[0], wr_ref[0], dims, preferred_element_type=jnp.float32))
    out = lax.dot_general(h, wo_ref[0], dims, preferred_element_type=jnp.float32) + bo_ref[0]
    out_ref[...] = jnp.maximum(out, 0.0)[None]


def _dense_stage(acc, cnt_t, x, wl, bl, wr, wo, bo):
    spec_rows = pl.BlockSpec((1, _BLK, _D), lambda t, i: (t, i, 0))
    spec_cnt = pl.BlockSpec((1, _BLK, _NS), lambda t, i: (t, i, 0))
    spec_w = pl.BlockSpec((1, _D, _D), lambda t, i: (t, 0, 0))
    spec_b = pl.BlockSpec((1, 1, _D), lambda t, i: (t, 0, 0))
    return pl.pallas_call(
        _dense_body,
        grid=(2, _NPAD // _BLK),
        in_specs=[spec_rows, spec_cnt, spec_rows, spec_w, spec_b, spec_w,
                  spec_w, spec_b],
        out_specs=spec_rows,
        out_shape=jax.ShapeDtypeStruct((2, _NPAD, _D), jnp.float32),
    )(acc, cnt_t, x, wl, bl, wr, wo, bo)


def _pad_edges(edge_index):
    src = edge_index[0].reshape(_NS, _E // _NS)
    dst = edge_index[1].reshape(_NS, _E // _NS)
    pad = _EPS - _E // _NS
    src = jnp.concatenate(
        [src, jnp.zeros((_NS, pad), jnp.int32)], axis=1).reshape(_NS, _CH, _CW)
    dst = jnp.concatenate(
        [dst, jnp.full((_NS, pad), _DUMMY, jnp.int32)], axis=1).reshape(_NS, _CH, _CW)
    return src, dst


def kernel(x_user, x_item, edge_index_u2i, edge_index_i2u,
           W_l_u2i, b_l_u2i, W_r_u2i, W_l_i2u, b_l_i2u, W_r_i2u,
           W_user, b_user, W_item, b_item):
    src_u2i, dst_u2i = _pad_edges(edge_index_u2i)
    src_i2u, dst_i2u = _pad_edges(edge_index_i2u)
    zeros = jnp.zeros((_RPS, _D), jnp.float32)
    zeros1d = jnp.zeros((_NPAD,), jnp.float32)

    acc, cnt = _sc_segment_sum()(x_user, x_item, src_u2i, dst_u2i,
                                 src_i2u, dst_i2u, zeros, zeros1d)

    cnt_t = jnp.transpose(cnt, (0, 2, 1))
    x = jnp.zeros((2, _NPAD, _D), jnp.float32).at[:, :_N, :].set(
        jnp.stack([x_item, x_user]))
    wl = jnp.stack([W_l_u2i, W_l_i2u])
    bl = jnp.stack([b_l_u2i, b_l_i2u])[:, None, :]
    wr = jnp.stack([W_r_u2i, W_r_i2u])
    wo = jnp.stack([W_item, W_user])
    bo = jnp.stack([b_item, b_user])[:, None, :]
    out = _dense_stage(acc, cnt_t, x, wl, bl, wr, wo, bo)
    return (out[1, :_N], out[0, :_N])
